# baseline (speedup 1.0000x reference)
import functools

import jax
import jax.numpy as jnp
from jax import lax
from jax.experimental import pallas as pl
from jax.experimental.pallas import tpu as pltpu
from jax.experimental.pallas import tpu_sc as plsc

_B = 16384
_D = 32
_NC = 2
_NS = 16
_NW = _NC * _NS
_BPW = _B // _NW
_V = 16

_mesh = plsc.VectorSubcoreMesh(core_axis_name="c", subcore_axis_name="s")


@functools.partial(
    pl.kernel,
    mesh=_mesh,
    out_type=jax.ShapeDtypeStruct((_B * _D,), jnp.float32),
    scratch_types=[
        pltpu.VMEM((_BPW,), jnp.int32),
        pltpu.VMEM((_BPW * _D,), jnp.int32),
        pltpu.VMEM((_BPW * _D,), jnp.float32),
        pltpu.SemaphoreType.DMA,
        pltpu.SemaphoreType.DMA,
    ],
)
def _sc_gather(flat_hbm, idx_hbm, out_hbm, idx_v, eidx_v, gath_v, sem_i, sem_g):
    wid = lax.axis_index("s") * _NC + lax.axis_index("c")
    base = wid * _BPW
    pltpu.async_copy(idx_hbm.at[pl.ds(base, _BPW)], idx_v, sem_i).wait()

    lane = lax.iota(jnp.int32, _V)

    @pl.loop(0, _BPW // _V)
    def expand(c):
        ivec = idx_v[pl.ds(c * _V, _V)] * _D
        for t in range(_V):
            s = ivec[t]
            for h in range(_D // _V):
                eidx_v[pl.ds((c * _V + t) * _D + h * _V, _V)] = s + h * _V + lane

    pltpu.async_copy(flat_hbm.at[eidx_v], gath_v, sem_g).wait()
    pltpu.sync_copy(gath_v, out_hbm.at[pl.ds(base * _D, _BPW * _D)])


def kernel(idx, representations):
    flat = representations.reshape(-1)
    out = _sc_gather(flat, idx.astype(jnp.int32))
    return out.reshape(_B, _D)
